# Initial kernel scaffold; baseline (speedup 1.0000x reference)
import jax
import jax.numpy as jnp
from jax.experimental import pallas as pl


def kernel(x, edge_index, edge_attr, batch, W1_rel, b1_rel, W1_root, W2_rel, b2_rel, W2_root, w_ih_l0f, w_hh_l0f, b_ih_l0f, b_hh_l0f, w_ih_l0b, w_hh_l0b, b_ih_l0b, b_hh_l0b, w_ih_l1f, w_hh_l1f, b_ih_l1f, b_hh_l1f, w_ih_l1b, w_hh_l1b, b_ih_l1b, b_hh_l1b, mlp_W1, mlp_b1, mlp_W2, mlp_b2):
    raise NotImplementedError("write your pallas kernel here")


# speedup vs baseline: 5.7461x; 5.7461x over previous
import functools

import jax
import jax.numpy as jnp
from jax import lax
from jax.experimental import pallas as pl
from jax.experimental.pallas import tpu as pltpu
from jax.experimental.pallas import tpu_sc as plsc

_N = 2048
_E = 32768
_BS = 64
_EC = 32
_DI = 7680
_DM = 4032
_DMP = 4096
_DO = 512
_H = 32
_NB = 1024
_EPT = _E // 16


def _adj_body(ei, ea, zeros_hbm, a_out, cnt_out,
              src_v, dst_v, ew_v, idx0, val0, idx1, val1, cidx, cval,
              a_sh, cnt_sh):
    c = lax.axis_index("c")
    s = lax.axis_index("s")
    base = s * _EPT
    pltpu.sync_copy(ei.at[0, pl.ds(base, _EPT)], src_v)
    pltpu.sync_copy(ei.at[1, pl.ds(base, _EPT)], dst_v)
    pltpu.sync_copy(ea.at[pl.ds(base, _EPT)], ew_v)
    row0 = c * _NB

    def scan_body(j, carry):
        for k in range(8):
            off = j * 128 + k * 16
            d = dst_v[pl.ds(off, 16)]
            sr = src_v[pl.ds(off, 16)]
            w = ew_v[pl.ds(off, 16)]
            rl = d - row0
            in_r = (rl >= 0) & (rl < _NB)
            cidx[j, pl.ds(k * 16, 16)] = jnp.where(in_r, rl, 0)
            cval[j, pl.ds(k * 16, 16)] = jnp.where(in_r, 1.0, 0.0)
            for h, (ib, vb) in enumerate(((idx0, val0), (idx1, val1))):
                cl = sr - h * _NB
                ok = in_r & (cl >= 0) & (cl < _NB)
                fi = rl * _NB + cl
                ib[j, pl.ds(k * 16, 16)] = jnp.where(ok, fi, 0)
                vb[j, pl.ds(k * 16, 16)] = jnp.where(ok, w, 0.0)
        return carry

    lax.fori_loop(0, 16, scan_body, 0)

    for h, (ib, vb) in enumerate(((idx0, val0), (idx1, val1))):
        pltpu.sync_copy(zeros_hbm, a_sh.at[pl.ds(s * (_NB * 64), _NB * 64)])
        if h == 0:
            @pl.when(s == 0)
            def _():
                pltpu.sync_copy(zeros_hbm.at[pl.ds(0, _NB)], cnt_sh)
        plsc.subcore_barrier()

        def sc_body(j, carry):
            pltpu.sync_copy(vb.at[j], a_sh.at[ib.at[j]], add=True)
            return carry

        lax.fori_loop(0, 16, sc_body, 0)
        if h == 0:
            def cnt_body(j, carry):
                pltpu.sync_copy(cval.at[j], cnt_sh.at[cidx.at[j]], add=True)
                return carry

            lax.fori_loop(0, 16, cnt_body, 0)
        plsc.subcore_barrier()
        pltpu.sync_copy(a_sh.at[pl.ds(s * (_NB * 64), _NB * 64)],
                        a_out.at[c, h, pl.ds(s * (_NB * 64), _NB * 64)])
        if h == 0:
            @pl.when(s == 0)
            def _():
                pltpu.sync_copy(cnt_sh, cnt_out.at[pl.ds(c * _NB, _NB)])
        plsc.subcore_barrier()


_build_adj = functools.partial(
    pl.kernel,
    out_type=(jax.ShapeDtypeStruct((2, 2, _NB * _NB), jnp.float32),
              jax.ShapeDtypeStruct((_N,), jnp.float32)),
    mesh=plsc.VectorSubcoreMesh(core_axis_name="c", subcore_axis_name="s"),
    scratch_types=[
        pltpu.VMEM((_EPT,), jnp.int32),
        pltpu.VMEM((_EPT,), jnp.int32),
        pltpu.VMEM((_EPT,), jnp.float32),
        pltpu.VMEM((16, 128), jnp.int32),
        pltpu.VMEM((16, 128), jnp.float32),
        pltpu.VMEM((16, 128), jnp.int32),
        pltpu.VMEM((16, 128), jnp.float32),
        pltpu.VMEM((16, 128), jnp.int32),
        pltpu.VMEM((16, 128), jnp.float32),
        pltpu.VMEM_SHARED((_NB * _NB,), jnp.float32),
        pltpu.VMEM_SHARED((_NB,), jnp.float32),
    ],
)(_adj_body)


def _mm_body(a_ref, b_ref, o_ref, acc_ref, *, nk):
    @pl.when(pl.program_id(2) == 0)
    def _():
        acc_ref[...] = jnp.zeros_like(acc_ref)

    acc_ref[...] += jnp.dot(a_ref[...], b_ref[...],
                            preferred_element_type=jnp.float32)

    @pl.when(pl.program_id(2) == nk - 1)
    def _():
        o_ref[...] = acc_ref[...]


def _matmul(a, b, bm, bn, bk):
    m, k = a.shape
    _, n = b.shape
    grid = (m // bm, n // bn, k // bk)
    return pl.pallas_call(
        functools.partial(_mm_body, nk=grid[2]),
        grid=grid,
        in_specs=[pl.BlockSpec((bm, bk), lambda i, j, kk: (i, kk)),
                  pl.BlockSpec((bk, bn), lambda i, j, kk: (kk, j))],
        out_specs=pl.BlockSpec((bm, bn), lambda i, j, kk: (i, j)),
        out_shape=jax.ShapeDtypeStruct((m, n), jnp.float32),
        scratch_shapes=[pltpu.VMEM((bm, bn), jnp.float32)],
        compiler_params=pltpu.CompilerParams(
            dimension_semantics=("parallel", "parallel", "arbitrary")),
    )(a, b)


def _cc_body(a_ref, xw_ref, xr_ref, cnt_ref, b_ref, o_ref, acc_ref, *, nk):
    @pl.when(pl.program_id(2) == 0)
    def _():
        acc_ref[...] = jnp.zeros_like(acc_ref)

    acc_ref[...] += jnp.dot(a_ref[0, 0], xw_ref[...],
                            preferred_element_type=jnp.float32)

    @pl.when(pl.program_id(2) == nk - 1)
    def _():
        cnt = jnp.maximum(cnt_ref[...], 1.0)
        o_ref[...] = jnp.maximum(
            acc_ref[...] / cnt + b_ref[...] + xr_ref[...], 0.0)


def _conv_combine(a_blocks, xwr, cnt2d, b2d, n_out, xr_blk0):
    bm, bn, bk = _NB, 512, _NB
    grid = (_N // bm, n_out // bn, 2)
    return pl.pallas_call(
        functools.partial(_cc_body, nk=grid[2]),
        grid=grid,
        in_specs=[
            pl.BlockSpec((1, 1, _NB, _NB), lambda i, j, kk: (i, kk, 0, 0)),
            pl.BlockSpec((bk, bn), lambda i, j, kk: (kk, j)),
            pl.BlockSpec((bm, bn), lambda i, j, kk: (i, xr_blk0 + j)),
            pl.BlockSpec((bm, 1), lambda i, j, kk: (i, 0)),
            pl.BlockSpec((1, bn), lambda i, j, kk: (0, j)),
        ],
        out_specs=pl.BlockSpec((bm, bn), lambda i, j, kk: (i, j)),
        out_shape=jax.ShapeDtypeStruct((_N, n_out), jnp.float32),
        scratch_shapes=[pltpu.VMEM((bm, bn), jnp.float32)],
        compiler_params=pltpu.CompilerParams(
            dimension_semantics=("parallel", "parallel", "arbitrary")),
    )(a_blocks, xwr, xwr, cnt2d, b2d)


def _lstm_body(seq_ref, wih0_ref, whh0_ref, b0_ref, wih1_ref, whh1_ref,
               b1_ref, w1r_ref, mb1_ref, w2_ref, mb2_ref, o_ref, y0_ref):
    T = _DO
    rmask = lax.broadcasted_iota(jnp.int32, (128, 1), 0) < 64

    def gates(xcat, hc, cc, wih, whh, bcat):
        p = (jnp.dot(xcat, wih, preferred_element_type=jnp.float32)
             + jnp.dot(hc, whh, preferred_element_type=jnp.float32))
        g = jnp.where(rmask, p[:, :128], p[:, 128:]) + bcat
        i_ = jax.nn.sigmoid(g[:, 0:32])
        f_ = jax.nn.sigmoid(g[:, 32:64])
        gg = jnp.tanh(g[:, 64:96])
        o_ = jax.nn.sigmoid(g[:, 96:128])
        cc = f_ * cc + i_ * gg
        hc = o_ * jnp.tanh(cc)
        return hc, cc

    def step0(u, carry):
        hc, cc = carry
        xf = seq_ref[pl.ds(u * 64, 64), :]
        xb = seq_ref[pl.ds((T - 1 - u) * 64, 64), :]
        hc, cc = gates(jnp.concatenate([xf, xb], axis=0), hc, cc,
                       wih0_ref[...], whh0_ref[...], b0_ref[...])
        y0_ref[pl.ds(u, 1), :, 0:32] = hc[0:64].reshape(1, 64, 32)
        y0_ref[pl.ds(T - 1 - u, 1), :, 32:64] = hc[64:128].reshape(1, 64, 32)
        return hc, cc

    z = jnp.zeros((128, 32), jnp.float32)
    lax.fori_loop(0, T, step0, (z, z))

    def step1(u, carry):
        hc, cc, macc = carry
        xf = y0_ref[pl.ds(u, 1)].reshape(64, 64)
        xb = y0_ref[pl.ds(T - 1 - u, 1)].reshape(64, 64)
        hc, cc = gates(jnp.concatenate([xf, xb], axis=0), hc, cc,
                       wih1_ref[...], whh1_ref[...], b1_ref[...])
        wf = w1r_ref[pl.ds(u, 1), 0:32, :].reshape(32, 128)
        wb = w1r_ref[pl.ds(T - 1 - u, 1), 32:64, :].reshape(32, 128)
        macc = (macc
                + jnp.dot(hc[0:64], wf, preferred_element_type=jnp.float32)
                + jnp.dot(hc[64:128], wb, preferred_element_type=jnp.float32))
        return hc, cc, macc

    _, _, macc = lax.fori_loop(0, T, step1,
                               (z, z, jnp.zeros((64, 128), jnp.float32)))
    m = jnp.maximum(macc + mb1_ref[...], 0.0)
    o_ref[...] = jax.nn.sigmoid(
        jnp.dot(m, w2_ref[...], preferred_element_type=jnp.float32)
        + mb2_ref[...])


def _lstm_mlp(seq2d, wih0, whh0, bcat0, wih1, whh1, bcat1, w1r, mb1, w2, mb2):
    return pl.pallas_call(
        _lstm_body,
        out_shape=jax.ShapeDtypeStruct((_BS, 1), jnp.float32),
        scratch_shapes=[pltpu.VMEM((_DO, _BS, 2 * _H), jnp.float32)],
    )(seq2d, wih0, whh0, bcat0, wih1, whh1, bcat1, w1r, mb1, w2, mb2)


def kernel(x, edge_index, edge_attr, batch, W1_rel, b1_rel, W1_root, W2_rel,
           b2_rel, W2_root, w_ih_l0f, w_hh_l0f, b_ih_l0f, b_hh_l0f, w_ih_l0b,
           w_hh_l0b, b_ih_l0b, b_hh_l0b, w_ih_l1f, w_hh_l1f, b_ih_l1f,
           b_hh_l1f, w_ih_l1b, w_hh_l1b, b_ih_l1b, b_hh_l1b, mlp_W1, mlp_b1,
           mlp_W2, mlp_b2):
    f32 = jnp.float32
    ei = edge_index.astype(jnp.int32)
    zeros_blk = jnp.zeros((_NB * 64,), f32)
    a_blocks, cnt = _build_adj(ei, edge_attr.astype(f32), zeros_blk)
    a_blocks = a_blocks.reshape(2, 2, _NB, _NB)
    cnt2d = cnt.reshape(_N, 1)

    W1cat = jnp.concatenate([jnp.pad(W1_rel, ((0, 0), (0, 64))),
                             jnp.pad(W1_root, ((0, 0), (0, 64)))], axis=1)
    b1p = jnp.pad(b1_rel, (0, 64)).reshape(1, _DMP)
    xwr1 = _matmul(x, W1cat, bm=1024, bn=1024, bk=768)
    h1 = _conv_combine(a_blocks, xwr1, cnt2d, b1p,
                       n_out=_DMP, xr_blk0=_DMP // 512)

    W2cat = jnp.concatenate([jnp.pad(W2_rel, ((0, 64), (0, 0))),
                             jnp.pad(W2_root, ((0, 64), (0, 0)))], axis=1)
    b2p = b2_rel.reshape(1, _DO)
    hwr2 = _matmul(h1, W2cat, bm=1024, bn=512, bk=1024)
    h2 = _conv_combine(a_blocks, hwr2, cnt2d, b2p,
                       n_out=_DO, xr_blk0=1)

    seq2d = h2.reshape(_BS, _EC, _DO).transpose(2, 0, 1).reshape(_DO * _BS, _EC)
    wih0 = jnp.concatenate([w_ih_l0f.T, w_ih_l0b.T], axis=1)
    whh0 = jnp.concatenate([w_hh_l0f.T, w_hh_l0b.T], axis=1)
    wih1 = jnp.concatenate([w_ih_l1f.T, w_ih_l1b.T], axis=1)
    whh1 = jnp.concatenate([w_hh_l1f.T, w_hh_l1b.T], axis=1)
    b0f = (b_ih_l0f + b_hh_l0f).reshape(1, 128)
    b0b = (b_ih_l0b + b_hh_l0b).reshape(1, 128)
    bcat0 = jnp.concatenate([jnp.tile(b0f, (64, 1)),
                             jnp.tile(b0b, (64, 1))], axis=0)
    b1f = (b_ih_l1f + b_hh_l1f).reshape(1, 128)
    b1b = (b_ih_l1b + b_hh_l1b).reshape(1, 128)
    bcat1 = jnp.concatenate([jnp.tile(b1f, (64, 1)),
                             jnp.tile(b1b, (64, 1))], axis=0)
    w1r = mlp_W1.reshape(_DO, 2 * _H, 128)
    return _lstm_mlp(seq2d, wih0, whh0, bcat0, wih1, whh1, bcat1, w1r,
                     mlp_b1.reshape(1, 128), mlp_W2, mlp_b2.reshape(1, 1))


# speedup vs baseline: 6.4457x; 1.1218x over previous
import functools

import jax
import jax.numpy as jnp
from jax import lax
from jax.experimental import pallas as pl
from jax.experimental.pallas import tpu as pltpu
from jax.experimental.pallas import tpu_sc as plsc

_N = 2048
_E = 32768
_BS = 64
_EC = 32
_DI = 7680
_DM = 4032
_DMP = 4096
_DO = 512
_H = 32
_NB = 1024
_EPT = _E // 16


def _adj_body(ei, ea, zeros_hbm, a_out, cnt_out,
              src_v, dst_v, ew_v, idx0, val0, idx1, val1, cidx, cval,
              a_sh, cnt_sh):
    c = lax.axis_index("c")
    s = lax.axis_index("s")
    base = s * _EPT
    pltpu.sync_copy(ei.at[0, pl.ds(base, _EPT)], src_v)
    pltpu.sync_copy(ei.at[1, pl.ds(base, _EPT)], dst_v)
    pltpu.sync_copy(ea.at[pl.ds(base, _EPT)], ew_v)
    row0 = c * _NB

    def scan_body(j, carry):
        for k in range(8):
            off = j * 128 + k * 16
            d = dst_v[pl.ds(off, 16)]
            sr = src_v[pl.ds(off, 16)]
            w = ew_v[pl.ds(off, 16)]
            rl = d - row0
            in_r = (rl >= 0) & (rl < _NB)
            cidx[j, pl.ds(k * 16, 16)] = jnp.where(in_r, rl, 0)
            cval[j, pl.ds(k * 16, 16)] = jnp.where(in_r, 1.0, 0.0)
            for h, (ib, vb) in enumerate(((idx0, val0), (idx1, val1))):
                cl = sr - h * _NB
                ok = in_r & (cl >= 0) & (cl < _NB)
                fi = rl * _NB + cl
                ib[j, pl.ds(k * 16, 16)] = jnp.where(ok, fi, 0)
                vb[j, pl.ds(k * 16, 16)] = jnp.where(ok, w, 0.0)
        return carry

    lax.fori_loop(0, 16, scan_body, 0)

    for h, (ib, vb) in enumerate(((idx0, val0), (idx1, val1))):
        pltpu.sync_copy(zeros_hbm, a_sh.at[pl.ds(s * (_NB * 64), _NB * 64)])
        if h == 0:
            @pl.when(s == 0)
            def _():
                pltpu.sync_copy(zeros_hbm.at[pl.ds(0, _NB)], cnt_sh)
        plsc.subcore_barrier()

        def sc_body(j, carry):
            pltpu.sync_copy(vb.at[j], a_sh.at[ib.at[j]], add=True)
            return carry

        lax.fori_loop(0, 16, sc_body, 0)
        if h == 0:
            def cnt_body(j, carry):
                pltpu.sync_copy(cval.at[j], cnt_sh.at[cidx.at[j]], add=True)
                return carry

            lax.fori_loop(0, 16, cnt_body, 0)
        plsc.subcore_barrier()
        pltpu.sync_copy(a_sh.at[pl.ds(s * (_NB * 64), _NB * 64)],
                        a_out.at[c, h, pl.ds(s * (_NB * 64), _NB * 64)])
        if h == 0:
            @pl.when(s == 0)
            def _():
                pltpu.sync_copy(cnt_sh, cnt_out.at[pl.ds(c * _NB, _NB)])
        plsc.subcore_barrier()


_build_adj = functools.partial(
    pl.kernel,
    out_type=(jax.ShapeDtypeStruct((2, 2, _NB * _NB), jnp.float32),
              jax.ShapeDtypeStruct((_N,), jnp.float32)),
    mesh=plsc.VectorSubcoreMesh(core_axis_name="c", subcore_axis_name="s"),
    scratch_types=[
        pltpu.VMEM((_EPT,), jnp.int32),
        pltpu.VMEM((_EPT,), jnp.int32),
        pltpu.VMEM((_EPT,), jnp.float32),
        pltpu.VMEM((16, 128), jnp.int32),
        pltpu.VMEM((16, 128), jnp.float32),
        pltpu.VMEM((16, 128), jnp.int32),
        pltpu.VMEM((16, 128), jnp.float32),
        pltpu.VMEM((16, 128), jnp.int32),
        pltpu.VMEM((16, 128), jnp.float32),
        pltpu.VMEM_SHARED((_NB * _NB,), jnp.float32),
        pltpu.VMEM_SHARED((_NB,), jnp.float32),
    ],
)(_adj_body)


def _mm_pad_body(a_ref, b_ref, o_ref, *, nk):
    @pl.when(pl.program_id(0) == 0)
    def _():
        o_ref[...] = jnp.zeros_like(o_ref)

    a = a_ref[...].astype(jnp.bfloat16)
    b = b_ref[...].astype(jnp.bfloat16)
    o_ref[:, 0:_DM] += jnp.dot(a, b, preferred_element_type=jnp.float32)


def _matmul_pad(a, b, bk):
    m, k = a.shape
    _, n = b.shape
    grid = (k // bk,)
    return pl.pallas_call(
        functools.partial(_mm_pad_body, nk=grid[0]),
        grid=grid,
        in_specs=[pl.BlockSpec((m, bk), lambda kk: (0, kk)),
                  pl.BlockSpec((bk, n), lambda kk: (kk, 0))],
        out_specs=pl.BlockSpec((m, _DMP), lambda kk: (0, 0)),
        out_shape=jax.ShapeDtypeStruct((m, _DMP), jnp.float32),
    )(a, b)


def _mm2_slice_body(a_ref, b_ref, b2_ref, o_ref, o2_ref):
    a = a_ref[:, 0:_DM]
    o_ref[...] = jnp.dot(a, b_ref[...], preferred_element_type=jnp.float32)
    o2_ref[...] = jnp.dot(a, b2_ref[...], preferred_element_type=jnp.float32)


def _matmul2_slice(a, b, b2, bm):
    m = a.shape[0]
    kb, n = b.shape
    grid = (m // bm,)
    bspec = pl.BlockSpec((kb, n), lambda i: (0, 0))
    ospec = pl.BlockSpec((bm, n), lambda i: (i, 0))
    return pl.pallas_call(
        _mm2_slice_body,
        grid=grid,
        in_specs=[pl.BlockSpec((bm, _DMP), lambda i: (i, 0)), bspec, bspec],
        out_specs=(ospec, ospec),
        out_shape=(jax.ShapeDtypeStruct((m, n), jnp.float32),
                   jax.ShapeDtypeStruct((m, n), jnp.float32)),
        compiler_params=pltpu.CompilerParams(
            dimension_semantics=("arbitrary",)),
    )(a, b, b2)


def _cc_body(a_ref, xw_ref, xr_ref, cnt_ref, b_ref, o_ref, acc_ref, *, nk,
             cdt):
    @pl.when(pl.program_id(2) == 0)
    def _():
        acc_ref[...] = jnp.zeros_like(acc_ref)

    acc_ref[...] += jnp.dot(a_ref[0, 0].astype(cdt), xw_ref[...].astype(cdt),
                            preferred_element_type=jnp.float32)

    @pl.when(pl.program_id(2) == nk - 1)
    def _():
        cnt = jnp.maximum(cnt_ref[...], 1.0)
        o_ref[...] = jnp.maximum(
            acc_ref[...] / cnt + b_ref[...] + xr_ref[...], 0.0)


def _conv_combine(a_blocks, xw, xr, cnt2d, b2d, bn, cdt=jnp.float32):
    n_out = xw.shape[1]
    bm, bk = _NB, _NB
    grid = (_N // bm, n_out // bn, 2)
    return pl.pallas_call(
        functools.partial(_cc_body, nk=grid[2], cdt=cdt),
        grid=grid,
        in_specs=[
            pl.BlockSpec((1, 1, _NB, _NB), lambda i, j, kk: (i, kk, 0, 0)),
            pl.BlockSpec((bk, bn), lambda i, j, kk: (kk, j)),
            pl.BlockSpec((bm, bn), lambda i, j, kk: (i, j)),
            pl.BlockSpec((bm, 1), lambda i, j, kk: (i, 0)),
            pl.BlockSpec((1, bn), lambda i, j, kk: (0, j)),
        ],
        out_specs=pl.BlockSpec((bm, bn), lambda i, j, kk: (i, j)),
        out_shape=jax.ShapeDtypeStruct((_N, n_out), jnp.float32),
        scratch_shapes=[pltpu.VMEM((bm, bn), jnp.float32)],
        compiler_params=pltpu.CompilerParams(
            dimension_semantics=("parallel", "parallel", "arbitrary")),
    )(a_blocks, xw, xr, cnt2d, b2d)


def _lstm_body(seq_ref, wih0_ref, whh0_ref, b0_ref, wih1_ref, whh1_ref,
               b1_ref, w1r_ref, mb1_ref, w2_ref, mb2_ref, o_ref, y0_ref):
    T = _DO
    rmask = lax.broadcasted_iota(jnp.int32, (128, 1), 0) < 64

    def gates(xcat, hc, cc, wih, whh, bcat):
        p = (jnp.dot(xcat, wih, preferred_element_type=jnp.float32)
             + jnp.dot(hc, whh, preferred_element_type=jnp.float32))
        g = jnp.where(rmask, p[:, :128], p[:, 128:]) + bcat
        i_ = jax.nn.sigmoid(g[:, 0:32])
        f_ = jax.nn.sigmoid(g[:, 32:64])
        gg = jnp.tanh(g[:, 64:96])
        o_ = jax.nn.sigmoid(g[:, 96:128])
        cc = f_ * cc + i_ * gg
        hc = o_ * jnp.tanh(cc)
        return hc, cc

    def step0(u, carry):
        hc, cc = carry
        xf = seq_ref[pl.ds(u * 64, 64), :]
        xb = seq_ref[pl.ds((T - 1 - u) * 64, 64), :]
        hc, cc = gates(jnp.concatenate([xf, xb], axis=0), hc, cc,
                       wih0_ref[...], whh0_ref[...], b0_ref[...])
        y0_ref[pl.ds(u, 1), :, 0:32] = hc[0:64].reshape(1, 64, 32)
        y0_ref[pl.ds(T - 1 - u, 1), :, 32:64] = hc[64:128].reshape(1, 64, 32)
        return hc, cc

    z = jnp.zeros((128, 32), jnp.float32)
    lax.fori_loop(0, T, step0, (z, z))

    def step1(u, carry):
        hc, cc, macc = carry
        xf = y0_ref[pl.ds(u, 1)].reshape(64, 64)
        xb = y0_ref[pl.ds(T - 1 - u, 1)].reshape(64, 64)
        hc, cc = gates(jnp.concatenate([xf, xb], axis=0), hc, cc,
                       wih1_ref[...], whh1_ref[...], b1_ref[...])
        wf = w1r_ref[pl.ds(u, 1), 0:32, :].reshape(32, 128)
        wb = w1r_ref[pl.ds(T - 1 - u, 1), 32:64, :].reshape(32, 128)
        macc = (macc
                + jnp.dot(hc[0:64], wf, preferred_element_type=jnp.float32)
                + jnp.dot(hc[64:128], wb, preferred_element_type=jnp.float32))
        return hc, cc, macc

    _, _, macc = lax.fori_loop(0, T, step1,
                               (z, z, jnp.zeros((64, 128), jnp.float32)))
    m = jnp.maximum(macc + mb1_ref[...], 0.0)
    o_ref[...] = jax.nn.sigmoid(
        jnp.dot(m, w2_ref[...], preferred_element_type=jnp.float32)
        + mb2_ref[...])


def _lstm_mlp(seq2d, wih0, whh0, bcat0, wih1, whh1, bcat1, w1r, mb1, w2, mb2):
    return pl.pallas_call(
        _lstm_body,
        out_shape=jax.ShapeDtypeStruct((_BS, 1), jnp.float32),
        scratch_shapes=[pltpu.VMEM((_DO, _BS, 2 * _H), jnp.float32)],
    )(seq2d, wih0, whh0, bcat0, wih1, whh1, bcat1, w1r, mb1, w2, mb2)


def kernel(x, edge_index, edge_attr, batch, W1_rel, b1_rel, W1_root, W2_rel,
           b2_rel, W2_root, w_ih_l0f, w_hh_l0f, b_ih_l0f, b_hh_l0f, w_ih_l0b,
           w_hh_l0b, b_ih_l0b, b_hh_l0b, w_ih_l1f, w_hh_l1f, b_ih_l1f,
           b_hh_l1f, w_ih_l1b, w_hh_l1b, b_ih_l1b, b_hh_l1b, mlp_W1, mlp_b1,
           mlp_W2, mlp_b2):
    f32 = jnp.float32
    ei = edge_index.astype(jnp.int32)
    zeros_blk = jnp.zeros((_NB * 64,), f32)
    a_blocks, cnt = _build_adj(ei, edge_attr.astype(f32), zeros_blk)
    a_blocks = a_blocks.reshape(2, 2, _NB, _NB)
    cnt2d = cnt.reshape(_N, 1)

    xw1 = _matmul_pad(x, W1_rel, bk=384)
    xr1 = _matmul_pad(x, W1_root, bk=384)
    b1p = jnp.pad(b1_rel, (0, _DMP - _DM)).reshape(1, _DMP)
    h1 = _conv_combine(a_blocks, xw1, xr1, cnt2d, b1p, bn=1024,
                       cdt=jnp.bfloat16)

    hw2, hr2 = _matmul2_slice(h1, W2_rel, W2_root, bm=512)
    h2 = _conv_combine(a_blocks, hw2, hr2, cnt2d,
                       b2_rel.reshape(1, _DO), bn=512)

    seq2d = h2.reshape(_BS, _EC, _DO).transpose(2, 0, 1).reshape(_DO * _BS, _EC)
    wih0 = jnp.concatenate([w_ih_l0f.T, w_ih_l0b.T], axis=1)
    whh0 = jnp.concatenate([w_hh_l0f.T, w_hh_l0b.T], axis=1)
    wih1 = jnp.concatenate([w_ih_l1f.T, w_ih_l1b.T], axis=1)
    whh1 = jnp.concatenate([w_hh_l1f.T, w_hh_l1b.T], axis=1)
    b0f = (b_ih_l0f + b_hh_l0f).reshape(1, 128)
    b0b = (b_ih_l0b + b_hh_l0b).reshape(1, 128)
    bcat0 = jnp.concatenate([jnp.tile(b0f, (64, 1)),
                             jnp.tile(b0b, (64, 1))], axis=0)
    b1f = (b_ih_l1f + b_hh_l1f).reshape(1, 128)
    b1b = (b_ih_l1b + b_hh_l1b).reshape(1, 128)
    bcat1 = jnp.concatenate([jnp.tile(b1f, (64, 1)),
                             jnp.tile(b1b, (64, 1))], axis=0)
    w1r = mlp_W1.reshape(_DO, 2 * _H, 128)
    return _lstm_mlp(seq2d, wih0, whh0, bcat0, wih1, whh1, bcat1, w1r,
                     mlp_b1.reshape(1, 128), mlp_W2, mlp_b2.reshape(1, 1))
